# baseline (speedup 1.0000x reference)
import functools
import jax
import jax.numpy as jnp
from jax import lax
from jax.experimental import pallas as pl
from jax.experimental.pallas import tpu as pltpu
from jax.experimental.pallas import tpu_sc as plsc


def kernel(data, emb):
    B, N, T, F = data.shape
    _, E = emb.shape
    W = F + E
    R = B * N
    NW = 32
    RPW = R // NW
    CHUNK = 5
    NCHUNK = RPW // CHUNK

    G = W // F
    flat = data.reshape(R, T, F)
    embv = emb.reshape(T, G - 1, F)
    mesh = plsc.VectorSubcoreMesh(
        core_axis_name="c", subcore_axis_name="s", num_cores=2, num_subcores=16
    )

    @functools.partial(
        pl.kernel,
        mesh=mesh,
        out_type=jax.ShapeDtypeStruct((R, T, G, F), jnp.float32),
        scratch_types=[pltpu.VMEM((CHUNK, T, G, F), jnp.float32)],
        compiler_params=pltpu.CompilerParams(use_tc_tiling_on_sc=False),
    )
    def sc_k(data_hbm, emb_hbm, out_hbm, outbuf):
        wid = lax.axis_index("s") * 2 + lax.axis_index("c")
        base = wid * RPW
        for c in range(CHUNK):
            pltpu.sync_copy(emb_hbm, outbuf.at[c, :, pl.ds(1, G - 1)])

        def body(g, carry):
            row = base + g * CHUNK
            pltpu.sync_copy(
                data_hbm.at[pl.ds(row, CHUNK)], outbuf.at[:, :, 0]
            )
            pltpu.sync_copy(outbuf, out_hbm.at[pl.ds(row, CHUNK)])
            return carry

        lax.fori_loop(0, NCHUNK, body, 0)

    out = sc_k(flat, embv)
    return out.reshape(B, N, T, W)


# speedup vs baseline: 42.5761x; 42.5761x over previous
import jax
import jax.numpy as jnp
from jax.experimental import pallas as pl


def _concat_kernel(d_ref, e_ref, o_ref):
    tt = d_ref.shape[1]
    f = d_ref.shape[2]
    e = e_ref.shape[1]
    n = d_ref.shape[3]
    o_ref[0, :, :f, :] = d_ref[0]
    for tl in range(tt):
        col = e_ref[0, :, tl]
        o_ref[0, tl, f:, :] = jnp.broadcast_to(col[:, None], (e, n))


def kernel(data, emb):
    B, N, T, F = data.shape
    _, E = emb.shape
    W = F + E
    TT = 8
    dataT = jnp.transpose(data, (0, 2, 3, 1))
    emb3 = emb.T.reshape(E, T // TT, TT).swapaxes(0, 1)

    out = pl.pallas_call(
        _concat_kernel,
        grid=(B, T // TT),
        in_specs=[
            pl.BlockSpec((1, TT, F, N), lambda b, t: (b, t, 0, 0)),
            pl.BlockSpec((1, E, TT), lambda b, t: (t, 0, 0)),
        ],
        out_specs=pl.BlockSpec((1, TT, W, N), lambda b, t: (b, t, 0, 0)),
        out_shape=jax.ShapeDtypeStruct((B, T, W, N), jnp.float32),
    )(dataT, emb3)
    return jnp.transpose(out, (0, 3, 1, 2))


# speedup vs baseline: 43.9375x; 1.0320x over previous
import jax
import jax.numpy as jnp
from jax.experimental import pallas as pl


def _concat_kernel(d_ref, e_ref, o_ref):
    tt = d_ref.shape[1]
    f = d_ref.shape[2]
    e = e_ref.shape[1]
    n = d_ref.shape[3]
    o_ref[0, :, :f, :] = d_ref[0]
    for tl in range(tt):
        col = e_ref[0, :, tl]
        o_ref[0, tl, f:, :] = jnp.broadcast_to(col[:, None], (e, n))


def kernel(data, emb):
    B, N, T, F = data.shape
    _, E = emb.shape
    W = F + E
    TT = 16
    dataT = jnp.transpose(data, (0, 2, 3, 1))
    emb3 = emb.T.reshape(E, T // TT, TT).swapaxes(0, 1)

    out = pl.pallas_call(
        _concat_kernel,
        grid=(B, T // TT),
        in_specs=[
            pl.BlockSpec((1, TT, F, N), lambda b, t: (b, t, 0, 0)),
            pl.BlockSpec((1, E, TT), lambda b, t: (t, 0, 0)),
        ],
        out_specs=pl.BlockSpec((1, TT, W, N), lambda b, t: (b, t, 0, 0)),
        out_shape=jax.ShapeDtypeStruct((B, T, W, N), jnp.float32),
    )(dataT, emb3)
    return jnp.transpose(out, (0, 3, 1, 2))
